# baseline (speedup 1.0000x reference)
import functools

import jax
import jax.numpy as jnp
from jax import lax
from jax.experimental import pallas as pl
from jax.experimental.pallas import tpu as pltpu
from jax.experimental.pallas import tpu_sc as plsc

B = 16384
EMB = 32
WIDE = 128
NC = 2
NS = 16
NW = NC * NS
CHUNK = 128
ROWS = B // CHUNK
RPW = ROWS // NW
NSTEP = 2 * RPW


def _gather_body(ur_hbm, mr_hbm, utab_hbm, mtab_hbm, uw_out, mw_out,
                 ur_v, mr_v, wide_v, gsem0, gsem1, wsem0, wsem1):
    wid = lax.axis_index("s") * NC + lax.axis_index("c")
    r0 = wid * RPW
    pltpu.sync_copy(ur_hbm.at[pl.ds(r0, RPW)], ur_v)
    pltpu.sync_copy(mr_hbm.at[pl.ds(r0, RPW)], mr_v)
    gsems = (gsem0, gsem1)
    wsems = (wsem0, wsem1)

    def step_src(s):
        if s < RPW:
            return utab_hbm.at[ur_v.at[s]]
        return mtab_hbm.at[mr_v.at[s - RPW]]

    def step_dst(s):
        j = s % RPW
        out = uw_out if s < RPW else mw_out
        return out.at[pl.ds(wid * RPW * CHUNK + j * CHUNK, CHUNK)]

    gathers = [None] * NSTEP
    writes = [None] * NSTEP
    gathers[0] = pltpu.async_copy(step_src(0), wide_v.at[0], gsems[0])
    for s in range(NSTEP):
        b = s % 2
        nb = (s + 1) % 2
        if s + 1 < NSTEP:
            if s >= 1:
                writes[s - 1].wait()
            gathers[s + 1] = pltpu.async_copy(step_src(s + 1),
                                              wide_v.at[nb], gsems[nb])
        gathers[s].wait()
        writes[s] = pltpu.async_copy(wide_v.at[b], step_dst(s), wsems[b])
    writes[NSTEP - 2].wait()
    writes[NSTEP - 1].wait()


@functools.partial(
    pl.kernel,
    mesh=plsc.VectorSubcoreMesh(core_axis_name="c", subcore_axis_name="s",
                                num_cores=NC),
    out_type=[
        jax.ShapeDtypeStruct((B, WIDE), jnp.float32),
        jax.ShapeDtypeStruct((B, WIDE), jnp.float32),
    ],
    scratch_types=[
        pltpu.VMEM((RPW, CHUNK), jnp.int32),
        pltpu.VMEM((RPW, CHUNK), jnp.int32),
        pltpu.VMEM((2, CHUNK, WIDE), jnp.float32),
        pltpu.SemaphoreType.DMA,
        pltpu.SemaphoreType.DMA,
        pltpu.SemaphoreType.DMA,
        pltpu.SemaphoreType.DMA,
    ],
)
def _gather(*args):
    _gather_body(*args)


def _mlp_body(uw_ref, mw_ref, uq_ref, mq_ref, w1a_ref, w1b_ref, b1_ref,
              w2_ref, b2_ref, w3_ref, b3_ref, out_ref):
    uq = uq_ref[...]
    mq = mq_ref[...]
    uw = uw_ref[...]
    mw = mw_ref[...]
    ue = jnp.zeros_like(uw[:, :EMB])
    me = jnp.zeros_like(ue)
    for k in range(4):
        ue = ue + uw[:, k * EMB:(k + 1) * EMB] * (uq == k).astype(jnp.float32)
        me = me + mw[:, k * EMB:(k + 1) * EMB] * (mq == k).astype(jnp.float32)
    x = jnp.dot(ue, w1a_ref[...], preferred_element_type=jnp.float32)
    x = x + jnp.dot(me, w1b_ref[...], preferred_element_type=jnp.float32)
    x = jnp.maximum(x + b1_ref[...], 0.0)
    x = jnp.maximum(
        jnp.dot(x, w2_ref[...], preferred_element_type=jnp.float32)
        + b2_ref[...], 0.0)
    out_ref[...] = (jnp.dot(x, w3_ref[...], preferred_element_type=jnp.float32)
                    + b3_ref[...])


def _mlp(uw, mw, uq, mq, W1a, W1b, b1, W2, b2, W3, b3):
    BB = 2048
    grid = (B // BB,)
    full = lambda shape: pl.BlockSpec(shape, lambda i: (0, 0))
    return pl.pallas_call(
        _mlp_body,
        grid=grid,
        in_specs=[
            pl.BlockSpec((BB, WIDE), lambda i: (i, 0)),
            pl.BlockSpec((BB, WIDE), lambda i: (i, 0)),
            pl.BlockSpec((BB, 1), lambda i: (i, 0)),
            pl.BlockSpec((BB, 1), lambda i: (i, 0)),
            full((EMB, 64)),
            full((EMB, 64)),
            full((1, 64)),
            full((64, 32)),
            full((1, 32)),
            full((32, 1)),
            full((1, 1)),
        ],
        out_specs=pl.BlockSpec((BB, 1), lambda i: (i, 0)),
        out_shape=jax.ShapeDtypeStruct((B, 1), jnp.float32),
    )(uw, mw, uq, mq, W1a, W1b, b1, W2, b2, W3, b3)


def kernel(user, movie, user_table, movie_table, W1, b1, W2, b2, W3, b3):
    user = user.astype(jnp.int32)
    movie = movie.astype(jnp.int32)
    ur = (user >> 2).reshape(ROWS, CHUNK)
    mr = (movie >> 2).reshape(ROWS, CHUNK)
    uq = (user & 3).reshape(B, 1)
    mq = (movie & 3).reshape(B, 1)
    utab_w = user_table.reshape(-1, WIDE)
    mtab_w = movie_table.reshape(-1, WIDE)
    uw, mw = _gather(ur, mr, utab_w, mtab_w)
    return _mlp(uw, mw, uq, mq, W1[:EMB], W1[EMB:], b1.reshape(1, 64),
                W2, b2.reshape(1, 32), W3, b3.reshape(1, 1))
